# baseline (speedup 1.0000x reference)
import jax
import jax.numpy as jnp
from jax.experimental import pallas as pl
from jax.experimental.pallas import tpu as pltpu

_BLK = 512
_K = 8
_W = 0.5
_EPS = 1e-8


def _prep_body(x_ref, fkc_ref, z_ref, s_ref):
    z = jnp.mean(x_ref[...], axis=0)
    z_ref[...] = z
    mu = jnp.mean(z, axis=0)
    zc = z - mu[None, :]
    cmat = jax.lax.dot_general(zc, zc, (((0,), (0,)), ((), ())),
                               preferred_element_type=jnp.float32)
    fro = jnp.sqrt(jnp.sum(cmat * cmat))
    fk = fkc_ref[0, 0] != 0.0
    fc = fkc_ref[0, 1] != 0.0
    both = fk & fc
    w1 = jnp.where(both, _W, jnp.where(fk, 1.0, 0.0))
    w2 = jnp.where(both, 1.0 - _W, jnp.where(fc, 1.0, 0.0))
    wi = jnp.where(fk | fc, 0.0, 1.0)
    lane = jax.lax.broadcasted_iota(jnp.int32, (1, 128), 1)
    out = jnp.where(lane == 0, fro, 0.0)
    out = jnp.where(lane == 1, w1, out)
    out = jnp.where(lane == 2, w2, out)
    out = jnp.where(lane == 3, wi, out)
    s_ref[...] = out


def _main_body(z_ref, zb_ref, s_ref, o_ref):
    i = pl.program_id(0)
    z = z_ref[...]
    zb = zb_ref[...]
    n = z.shape[0]
    blk = zb.shape[0]

    g = jax.lax.dot_general(zb, z, (((1,), (1,)), ((), ())),
                            preferred_element_type=jnp.float32)
    mu = jnp.mean(z, axis=0)
    u = jnp.sum(z * mu[None, :], axis=1)
    ub = jnp.sum(zb * mu[None, :], axis=1)
    c = jnp.sum(mu * mu)
    zz = jnp.sum(z * z, axis=1)
    invfe = 1.0 / (s_ref[0, 0] + _EPS)

    k2 = g + (-ub)[:, None] + (c - u)[None, :]
    l1 = jnp.sum(jnp.abs(k2), axis=1) * invfe + _EPS
    s = 0.01 + _EPS

    h = g - 0.5 * zz[None, :]
    m = jnp.max(h, axis=1)
    for _ in range(_K - 1):
        m = jnp.max(jnp.where(h < m[:, None], h, -jnp.inf), axis=1)
    w = s_ref[0, 1:4]
    c1 = w[0] / (float(_K) + _EPS)
    c2 = (w[1] * invfe) / (l1 * s)
    c3 = w[1] * 0.01 / s + w[2]

    c2k2 = c2[:, None] * k2
    aw = jnp.where(h >= m[:, None], c2k2 + c1, c2k2)
    zzb = jnp.sum(zb * zb, axis=1)
    awd = c1 + c2 * (zzb - 2.0 * ub + c)
    sumaw = jnp.sum(jnp.abs(aw), axis=1)
    den = jnp.maximum(sumaw - jnp.abs(awd) + jnp.abs(awd + c3), 1e-12)
    invden = 1.0 / den
    o_ref[...] = aw * invden[:, None]
    col = jax.lax.broadcasted_iota(jnp.int32, (blk, blk), 1)
    row = jax.lax.broadcasted_iota(jnp.int32, (blk, blk), 0)
    eye = (col == row).astype(jnp.float32)
    dcols = pl.ds(i * blk, blk)
    o_ref[:, dcols] = o_ref[:, dcols] + (c3 * invden)[:, None] * eye


def kernel(x, f_k, f_c):
    b, n, d = x.shape
    nb = n // _BLK

    fkc = jnp.stack([jnp.asarray(f_k), jnp.asarray(f_c)]) \
        .astype(jnp.float32).reshape(1, 2)
    z, sv = pl.pallas_call(
        _prep_body,
        out_shape=[
            jax.ShapeDtypeStruct((n, d), jnp.float32),
            jax.ShapeDtypeStruct((1, 128), jnp.float32),
        ],
    )(x, fkc)

    out = pl.pallas_call(
        _main_body,
        grid=(nb,),
        in_specs=[
            pl.BlockSpec((n, d), lambda i: (0, 0)),
            pl.BlockSpec((_BLK, d), lambda i: (i, 0)),
            pl.BlockSpec((1, 128), lambda i: (0, 0)),
        ],
        out_specs=pl.BlockSpec((_BLK, n), lambda i: (i, 0)),
        out_shape=jax.ShapeDtypeStruct((n, n), jnp.float32),
        compiler_params=pltpu.CompilerParams(
            dimension_semantics=("parallel",)),
    )(z, z, sv)
    return out


# speedup vs baseline: 1.0710x; 1.0710x over previous
import jax
import jax.numpy as jnp
from jax.experimental import pallas as pl
from jax.experimental.pallas import tpu as pltpu

_BLK = 512
_K = 8
_W = 0.5
_EPS = 1e-8


def _prep_body(x_ref, fkc_ref, z_ref, s_ref):
    z = jnp.mean(x_ref[...], axis=0)
    z_ref[...] = z
    mu = jnp.mean(z, axis=0)
    zc = z - mu[None, :]
    cmat = jax.lax.dot_general(zc, zc, (((0,), (0,)), ((), ())),
                               preferred_element_type=jnp.float32)
    fro = jnp.sqrt(jnp.sum(cmat * cmat))
    fk = fkc_ref[0, 0] != 0.0
    fc = fkc_ref[0, 1] != 0.0
    both = fk & fc
    w1 = jnp.where(both, _W, jnp.where(fk, 1.0, 0.0))
    w2 = jnp.where(both, 1.0 - _W, jnp.where(fc, 1.0, 0.0))
    wi = jnp.where(fk | fc, 0.0, 1.0)
    lane = jax.lax.broadcasted_iota(jnp.int32, (1, 128), 1)
    out = jnp.where(lane == 0, fro, 0.0)
    out = jnp.where(lane == 1, w1, out)
    out = jnp.where(lane == 2, w2, out)
    out = jnp.where(lane == 3, wi, out)
    s_ref[...] = out


def _main_body(z_ref, zb_ref, s_ref, o_ref):
    i = pl.program_id(0)
    z = z_ref[...]
    zb = zb_ref[...]
    n = z.shape[0]
    blk = zb.shape[0]

    g = jax.lax.dot_general(zb, z, (((1,), (1,)), ((), ())),
                            preferred_element_type=jnp.float32)
    mu = jnp.mean(z, axis=0)
    u = jnp.sum(z * mu[None, :], axis=1)
    ub = jnp.sum(zb * mu[None, :], axis=1)
    c = jnp.sum(mu * mu)
    zz = jnp.sum(z * z, axis=1)
    invfe = 1.0 / (s_ref[0, 0] + _EPS)

    ones = jnp.ones((n, 1), jnp.float32)

    k2 = g + (-ub)[:, None] + (c - u)[None, :]
    absk2 = jnp.abs(k2)
    l1 = jax.lax.dot_general(absk2, ones, (((1,), (0,)), ((), ())),
                             preferred_element_type=jnp.float32)[:, 0] \
        * invfe + _EPS
    s = 0.01 + _EPS

    h = g - 0.5 * zz[None, :]
    m = jnp.max(h, axis=1)
    for _ in range(_K - 1):
        m = jnp.max(jnp.where(h < m[:, None], h, -jnp.inf), axis=1)
    w = s_ref[0, 1:4]
    c1 = w[0] / (float(_K) + _EPS)
    c2 = (w[1] * invfe) / (l1 * s)
    c3 = w[1] * 0.01 / s + w[2]

    c2k2 = c2[:, None] * k2
    aw = jnp.where(h >= m[:, None], c2k2 + c1, c2k2)
    zzb = jnp.sum(zb * zb, axis=1)
    awd = c1 + c2 * (zzb - 2.0 * ub + c)
    absaw = jnp.abs(aw)
    sumaw = jax.lax.dot_general(absaw, ones, (((1,), (0,)), ((), ())),
                                preferred_element_type=jnp.float32)[:, 0]
    den = jnp.maximum(sumaw - jnp.abs(awd) + jnp.abs(awd + c3), 1e-12)
    invden = 1.0 / den
    o_ref[...] = aw * invden[:, None]
    col = jax.lax.broadcasted_iota(jnp.int32, (blk, blk), 1)
    row = jax.lax.broadcasted_iota(jnp.int32, (blk, blk), 0)
    eye = (col == row).astype(jnp.float32)
    dcols = pl.ds(i * blk, blk)
    o_ref[:, dcols] = o_ref[:, dcols] + (c3 * invden)[:, None] * eye


def kernel(x, f_k, f_c):
    b, n, d = x.shape
    nb = n // _BLK

    fkc = jnp.stack([jnp.asarray(f_k), jnp.asarray(f_c)]) \
        .astype(jnp.float32).reshape(1, 2)
    z, sv = pl.pallas_call(
        _prep_body,
        out_shape=[
            jax.ShapeDtypeStruct((n, d), jnp.float32),
            jax.ShapeDtypeStruct((1, 128), jnp.float32),
        ],
    )(x, fkc)

    out = pl.pallas_call(
        _main_body,
        grid=(nb,),
        in_specs=[
            pl.BlockSpec((n, d), lambda i: (0, 0)),
            pl.BlockSpec((_BLK, d), lambda i: (i, 0)),
            pl.BlockSpec((1, 128), lambda i: (0, 0)),
        ],
        out_specs=pl.BlockSpec((_BLK, n), lambda i: (i, 0)),
        out_shape=jax.ShapeDtypeStruct((n, n), jnp.float32),
        compiler_params=pltpu.CompilerParams(
            dimension_semantics=("parallel",)),
    )(z, z, sv)
    return out
